# baseline (speedup 1.0000x reference)
import functools

import numpy as np
import jax
import jax.numpy as jnp
from jax import lax
from jax.experimental import pallas as pl
from jax.experimental.pallas import tpu as pltpu
from jax.experimental.pallas import tpu_sc as plsc

N = 10000
E = 320000
D = 128
H = 8
DH = D // H
SCALE = DH ** -0.5

NC = 2
NS = 16
NW = NC * NS
PER_W = E // NW
CB = 80
NCH = PER_W // CB
NPAD = 10240
ROWS_PER_SUB = NPAD // NS

EBLK = 2560
GE = E // EBLK
NB = 400
GN = N // NB

_EXP_NP = np.kron(np.eye(H, dtype=np.float32), np.ones((1, DH), np.float32))


def _ln(x, g, b, eps=1e-5):
    m = jnp.mean(x, axis=-1, keepdims=True)
    v = jnp.mean((x - m) ** 2, axis=-1, keepdims=True)
    return (x - m) / jnp.sqrt(v + eps) * g + b


def _gelu(x):
    return x * 0.5 * (1.0 + lax.erf(x * np.float32(1.0 / np.sqrt(2.0))))


def _dot(a, b):
    return jnp.dot(a, b, preferred_element_type=jnp.float32)


def _tc_qkv_body(x_ref, wq, bq, wk, bk, wv, bv, q_out, k_out, v_out):
    x = x_ref[...]
    q_out[...] = _dot(x, wq[...]) + bq[...]
    k_out[...] = _dot(x, wk[...]) + bk[...]
    v_out[...] = _dot(x, wv[...]) + bv[...]


def _tc_eb_body(he_ref, w, b, eb_out):
    eb_out[...] = _dot(he_ref[...], w[...]) + b[...]


def _tc_msg_body(qd, ks, eb, hs, expm, exx_out):
    s = _dot(qd[...] * ks[...], hs[...]) * SCALE + eb[...]
    exx_out[...] = _dot(jnp.exp(s), expm[...])


def _tc_node_body(hnode, on0, on1, den0, den1,
                  wo, bo, g1, b1, wf1, bf1, wf2, bf2, g2, b2,
                  w1a, b1u, w1b, hn_out, a_out, b_out):
    agg = (on0[...] + on1[...]) / (den0[...] + den1[...] + 1e-16)
    out_node = _dot(agg, wo[...]) + bo[...]
    h1 = _ln(hnode[...] + out_node, g1[...], b1[...])
    ff = _dot(_gelu(_dot(h1, wf1[...]) + bf1[...]), wf2[...]) + bf2[...]
    hn = _ln(h1 + ff, g2[...], b2[...])
    hn_out[...] = hn
    a_out[...] = _dot(hn, w1a[...]) + b1u[...]
    b_out[...] = _dot(hn, w1b[...])


def _tc_edge_body(anbn, he, w1c, w2, b2, ge, be, he_out):
    t = anbn[...] + _dot(he[...], w1c[...])
    hen = _dot(_gelu(t), w2[...]) + b2[...]
    he_out[...] = _ln(he[...] + hen, ge[...], be[...])


def _pipeline(nch, fire, complete, wait_reuse):
    p = nch // 2
    odd = nch % 2 == 1
    fire(0, 0)
    fire(1, 1)

    def body(j, carry):
        c0 = 2 * j
        complete(c0, 0)
        complete(c0 + 1, 1)
        wait_reuse(c0, 0)
        if odd:
            fire(c0 + 2, 0)
        else:
            @pl.when(j < p - 1)
            def _pf0():
                fire(c0 + 2, 0)
        wait_reuse(c0 + 1, 1)

        @pl.when(j < p - 1)
        def _pf1():
            fire(c0 + 3, 1)

        return carry

    lax.fori_loop(0, p, body, 0)
    if odd:
        complete(nch - 1, 0)
        wait_reuse(nch - 1, 0)


def _vec_binop(dst, src, op):
    def row(i, carry):
        for r in range(D // 16):
            sl = pl.ds(r * 16, 16)
            dst[i, sl] = op(dst[i, sl], src[i, sl])
        return carry

    lax.fori_loop(0, CB, row, 0)


def _make_gather(mesh, use_dst, combine):
    n = len(use_dst)
    n_out = 1 if combine else n

    @functools.partial(
        pl.kernel,
        mesh=mesh,
        out_type=[jax.ShapeDtypeStruct((E, D), jnp.float32)] * n_out,
        scratch_types=(
            [pltpu.VMEM((CB,), jnp.int32)] * 4
            + [pltpu.VMEM((CB, D), jnp.float32)] * (2 * n)
            + [pltpu.SemaphoreType.DMA] * 4
        ),
    )
    def gather(*refs):
        tabs = refs[:n]
        src_hbm, dst_hbm = refs[n], refs[n + 1]
        outs = refs[n + 2:n + 2 + n_out]
        scr = refs[n + 2 + n_out:]
        idx = (scr[0:2], scr[2:4])
        bufs = (scr[4:4 + n], scr[4 + n:4 + 2 * n])
        sem_g = scr[4 + 2 * n:6 + 2 * n]
        sem_w = scr[6 + 2 * n:8 + 2 * n]

        wid = lax.axis_index("s") * NC + lax.axis_index("c")
        base = wid * PER_W

        def gidx(slot, t):
            return idx[slot][1] if use_dst[t] else idx[slot][0]

        def fire(c, slot):
            off = base + c * CB
            pltpu.sync_copy(src_hbm.at[pl.ds(off, CB)], idx[slot][0])
            pltpu.sync_copy(dst_hbm.at[pl.ds(off, CB)], idx[slot][1])
            for t in range(n):
                pltpu.async_copy(tabs[t].at[gidx(slot, t)], bufs[slot][t],
                                 sem_g[slot])

        def complete(c, slot):
            off = base + c * CB
            for t in range(n):
                pltpu.make_async_copy(tabs[t].at[gidx(slot, t)],
                                      bufs[slot][t], sem_g[slot]).wait()
            if combine:
                for t in range(1, n):
                    _vec_binop(bufs[slot][0], bufs[slot][t],
                               lambda a, b: a + b)
            for t in range(n_out):
                pltpu.async_copy(bufs[slot][t], outs[t].at[pl.ds(off, CB)],
                                 sem_w[slot])

        def wait_reuse(c, slot):
            off = base + c * CB
            for t in range(n_out):
                pltpu.make_async_copy(bufs[slot][t],
                                      outs[t].at[pl.ds(off, CB)],
                                      sem_w[slot]).wait()

        _pipeline(NCH, fire, complete, wait_reuse)

    return gather


@functools.cache
def _sc_kernels():
    mesh = plsc.VectorSubcoreMesh(core_axis_name="c", subcore_axis_name="s")

    gather_qk = _make_gather(mesh, (True, False), combine=False)
    gather_ab = _make_gather(mesh, (False, True), combine=True)

    @functools.partial(
        pl.kernel,
        mesh=mesh,
        out_type=[jax.ShapeDtypeStruct((NC, NPAD, D), jnp.float32)] * 2,
        scratch_types=(
            [pltpu.VMEM((CB,), jnp.int32)] * 4
            + [pltpu.VMEM((CB, D), jnp.float32)] * 4
            + [pltpu.VMEM_SHARED((NPAD, D), jnp.float32)]
            + [pltpu.SemaphoreType.DMA] * 2
        ),
    )
    def scatter(src_hbm, dst_hbm, v_hbm, exx_hbm, zero_hbm, on_out, den_out,
                is0, is1, id0, id1, bv0, bv1, be0, be1, sh, sem0, sem1):
        cid = lax.axis_index("c")
        sid = lax.axis_index("s")
        wid = sid * NC + cid
        r0 = sid * ROWS_PER_SUB
        base = wid * PER_W
        idx_s = (is0, is1)
        idx_d = (id0, id1)
        bv = (bv0, bv1)
        be = (be0, be1)
        sems = (sem0, sem1)

        def zero_init():
            pltpu.sync_copy(zero_hbm.at[pl.ds(r0, ROWS_PER_SUB)],
                            sh.at[pl.ds(r0, ROWS_PER_SUB)])
            plsc.subcore_barrier()

        def drain(out_hbm):
            plsc.subcore_barrier()
            pltpu.sync_copy(sh.at[pl.ds(r0, ROWS_PER_SUB)],
                            out_hbm.at[cid, pl.ds(r0, ROWS_PER_SUB)])
            plsc.subcore_barrier()

        zero_init()

        def fire1(c, slot):
            off = base + c * CB
            pltpu.sync_copy(src_hbm.at[pl.ds(off, CB)], idx_s[slot])
            pltpu.sync_copy(dst_hbm.at[pl.ds(off, CB)], idx_d[slot])
            pltpu.async_copy(v_hbm.at[idx_s[slot]], bv[slot], sems[slot])
            pltpu.async_copy(exx_hbm.at[pl.ds(off, CB)], be[slot],
                             sems[slot])

        def complete1(c, slot):
            off = base + c * CB
            pltpu.make_async_copy(v_hbm.at[idx_s[slot]], bv[slot],
                                  sems[slot]).wait()
            pltpu.make_async_copy(exx_hbm.at[pl.ds(off, CB)], be[slot],
                                  sems[slot]).wait()
            _vec_binop(bv[slot], be[slot], lambda a, b: a * b)
            pltpu.sync_copy(bv[slot], sh.at[idx_d[slot]], add=True)

        def noop(c, slot):
            pass

        _pipeline(NCH, fire1, complete1, noop)
        drain(on_out)

        zero_init()

        def fire2(c, slot):
            off = base + c * CB
            pltpu.async_copy(dst_hbm.at[pl.ds(off, CB)], idx_d[slot],
                             sems[slot])
            pltpu.async_copy(exx_hbm.at[pl.ds(off, CB)], be[slot],
                             sems[slot])

        def complete2(c, slot):
            off = base + c * CB
            pltpu.make_async_copy(dst_hbm.at[pl.ds(off, CB)], idx_d[slot],
                                  sems[slot]).wait()
            pltpu.make_async_copy(exx_hbm.at[pl.ds(off, CB)], be[slot],
                                  sems[slot]).wait()
            pltpu.sync_copy(be[slot], sh.at[idx_d[slot]], add=True)

        _pipeline(NCH, fire2, complete2, noop)
        drain(den_out)

    return gather_qk, gather_ab, scatter


def kernel(h_node, h_edge, edge_index, params):
    p = params
    gqk, gab, sct = _sc_kernels()
    ei = edge_index.astype(jnp.int32)
    e_src, e_dst = ei[0], ei[1]
    expm = jnp.asarray(_EXP_NP)
    hs = jnp.asarray(_EXP_NP.T)
    r2 = lambda t: t.reshape(1, -1)
    wspec = lambda shp: pl.BlockSpec(shp, lambda i: (0, 0))

    q, k, v = pl.pallas_call(
        _tc_qkv_body,
        out_shape=[jax.ShapeDtypeStruct((N, D), jnp.float32)] * 3,
    )(h_node, p["q"]["W"], r2(p["q"]["b"]), p["k"]["W"], r2(p["k"]["b"]),
      p["v"]["W"], r2(p["v"]["b"]))

    eb = pl.pallas_call(
        _tc_eb_body,
        grid=(GE,),
        in_specs=[pl.BlockSpec((EBLK, D), lambda i: (i, 0)),
                  pl.BlockSpec((D, H), lambda i: (0, 0)),
                  pl.BlockSpec((1, H), lambda i: (0, 0))],
        out_specs=pl.BlockSpec((EBLK, H), lambda i: (i, 0)),
        out_shape=jax.ShapeDtypeStruct((E, H), jnp.float32),
    )(h_edge, p["eb"]["W"], r2(p["eb"]["b"]))

    qd, ks = gqk(q, k, e_src, e_dst)

    exx = pl.pallas_call(
        _tc_msg_body,
        grid=(GE,),
        in_specs=[pl.BlockSpec((EBLK, D), lambda i: (i, 0)),
                  pl.BlockSpec((EBLK, D), lambda i: (i, 0)),
                  pl.BlockSpec((EBLK, H), lambda i: (i, 0)),
                  pl.BlockSpec((D, H), lambda i: (0, 0)),
                  pl.BlockSpec((H, D), lambda i: (0, 0))],
        out_specs=pl.BlockSpec((EBLK, D), lambda i: (i, 0)),
        out_shape=jax.ShapeDtypeStruct((E, D), jnp.float32),
    )(qd, ks, eb, hs, expm)

    zero = jnp.zeros((NPAD, D), jnp.float32)
    on_p, den_p = sct(e_src, e_dst, v, exx, zero)

    hn, a_tab, b_tab = pl.pallas_call(
        _tc_node_body,
        grid=(GN,),
        in_specs=[pl.BlockSpec((NB, D), lambda i: (i, 0))] * 5 +
                 [wspec((D, D)), wspec((1, D)),
                  wspec((1, D)), wspec((1, D)),
                  wspec((D, 2 * D)), wspec((1, 2 * D)),
                  wspec((2 * D, D)), wspec((1, D)),
                  wspec((1, D)), wspec((1, D)),
                  wspec((D, D)), wspec((1, D)), wspec((D, D))],
        out_specs=[pl.BlockSpec((NB, D), lambda i: (i, 0))] * 3,
        out_shape=[jax.ShapeDtypeStruct((N, D), jnp.float32)] * 3,
    )(h_node, on_p[0], on_p[1], den_p[0], den_p[1],
      p["o"]["W"], r2(p["o"]["b"]),
      r2(p["ln1"]["g"]), r2(p["ln1"]["b"]),
      p["ffn1"]["W"], r2(p["ffn1"]["b"]),
      p["ffn2"]["W"], r2(p["ffn2"]["b"]),
      r2(p["ln2"]["g"]), r2(p["ln2"]["b"]),
      p["eu1"]["W"][:D], r2(p["eu1"]["b"]), p["eu1"]["W"][D:2 * D])

    res = gab(a_tab, b_tab, e_src, e_dst)
    anbn = res[0] if isinstance(res, (list, tuple)) else res

    he = pl.pallas_call(
        _tc_edge_body,
        grid=(GE,),
        in_specs=[pl.BlockSpec((EBLK, D), lambda i: (i, 0)),
                  pl.BlockSpec((EBLK, D), lambda i: (i, 0)),
                  wspec((D, D)), wspec((D, D)), wspec((1, D)),
                  wspec((1, D)), wspec((1, D))],
        out_specs=pl.BlockSpec((EBLK, D), lambda i: (i, 0)),
        out_shape=jax.ShapeDtypeStruct((E, D), jnp.float32),
    )(anbn, h_edge, p["eu1"]["W"][2 * D:], p["eu2"]["W"], r2(p["eu2"]["b"]),
      r2(p["lne"]["g"]), r2(p["lne"]["b"]))

    return hn, he


# speedup vs baseline: 1.0256x; 1.0256x over previous
import functools

import numpy as np
import jax
import jax.numpy as jnp
from jax import lax
from jax.experimental import pallas as pl
from jax.experimental.pallas import tpu as pltpu
from jax.experimental.pallas import tpu_sc as plsc

N = 10000
E = 320000
D = 128
H = 8
DH = D // H
SCALE = DH ** -0.5

NC = 2
NS = 16
NW = NC * NS
PER_W = E // NW
CB = 80
NCH = PER_W // CB
NPAD = 10240
ROWS_PER_SUB = NPAD // NS

EBLK = 2560
GE = E // EBLK
NB = 400
GN = N // NB

_EXP_NP = np.kron(np.eye(H, dtype=np.float32), np.ones((1, DH), np.float32))


def _ln(x, g, b, eps=1e-5):
    m = jnp.mean(x, axis=-1, keepdims=True)
    v = jnp.mean((x - m) ** 2, axis=-1, keepdims=True)
    return (x - m) / jnp.sqrt(v + eps) * g + b


def _gelu(x):
    return x * 0.5 * (1.0 + lax.erf(x * np.float32(1.0 / np.sqrt(2.0))))


def _dot(a, b):
    return jnp.dot(a, b, preferred_element_type=jnp.float32)


def _tc_qkv_body(x_ref, wq, bq, wk, bk, wv, bv, q_out, k_out, v_out):
    x = x_ref[...]
    q_out[...] = _dot(x, wq[...]) + bq[...]
    k_out[...] = _dot(x, wk[...]) + bk[...]
    v_out[...] = _dot(x, wv[...]) + bv[...]


def _tc_eb_body(he_ref, w, b, eb_out):
    eb_out[...] = _dot(he_ref[...], w[...]) + b[...]


def _tc_msg_body(qd, ks, eb, hs, expm, exx_out):
    s = _dot(qd[...] * ks[...], hs[...]) * SCALE + eb[...]
    exx_out[...] = _dot(jnp.exp(s), expm[...])


def _tc_node_body(hnode, on0, on1, den0, den1,
                  wo, bo, g1, b1, wf1, bf1, wf2, bf2, g2, b2,
                  w1a, b1u, w1b, hn_out, a_out, b_out):
    agg = (on0[...] + on1[...]) / (den0[...] + den1[...] + 1e-16)
    out_node = _dot(agg, wo[...]) + bo[...]
    h1 = _ln(hnode[...] + out_node, g1[...], b1[...])
    ff = _dot(_gelu(_dot(h1, wf1[...]) + bf1[...]), wf2[...]) + bf2[...]
    hn = _ln(h1 + ff, g2[...], b2[...])
    hn_out[...] = hn
    a_out[...] = _dot(hn, w1a[...]) + b1u[...]
    b_out[...] = _dot(hn, w1b[...])


def _tc_edge_body(anbn, he, w1c, w2, b2, ge, be, he_out):
    t = anbn[...] + _dot(he[...], w1c[...])
    hen = _dot(_gelu(t), w2[...]) + b2[...]
    he_out[...] = _ln(he[...] + hen, ge[...], be[...])


def _pipeline(nch, fire, complete, wait_reuse):
    p = nch // 2
    odd = nch % 2 == 1
    fire(0, 0)
    fire(1, 1)

    def body(j, carry):
        c0 = 2 * j
        complete(c0, 0)
        complete(c0 + 1, 1)
        wait_reuse(c0, 0)
        if odd:
            fire(c0 + 2, 0)
        else:
            @pl.when(j < p - 1)
            def _pf0():
                fire(c0 + 2, 0)
        wait_reuse(c0 + 1, 1)

        @pl.when(j < p - 1)
        def _pf1():
            fire(c0 + 3, 1)

        return carry

    lax.fori_loop(0, p, body, 0)
    if odd:
        complete(nch - 1, 0)
        wait_reuse(nch - 1, 0)


def _vec_binop(dst, src, op):
    def row(i, carry):
        for r in range(D // 16):
            sl = pl.ds(r * 16, 16)
            dst[i, sl] = op(dst[i, sl], src[i, sl])
        return carry

    lax.fori_loop(0, CB, row, 0)


def _make_gather(mesh, use_dst, combine):
    n = len(use_dst)
    n_out = 1 if combine else n

    @functools.partial(
        pl.kernel,
        mesh=mesh,
        out_type=[jax.ShapeDtypeStruct((E, D), jnp.float32)] * n_out,
        scratch_types=(
            [pltpu.VMEM((CB,), jnp.int32)] * 4
            + [pltpu.VMEM((CB, D), jnp.float32)] * (2 * n)
            + [pltpu.SemaphoreType.DMA] * 4
        ),
    )
    def gather(*refs):
        tabs = refs[:n]
        src_hbm, dst_hbm = refs[n], refs[n + 1]
        outs = refs[n + 2:n + 2 + n_out]
        scr = refs[n + 2 + n_out:]
        idx = (scr[0:2], scr[2:4])
        bufs = (scr[4:4 + n], scr[4 + n:4 + 2 * n])
        sem_g = scr[4 + 2 * n:6 + 2 * n]
        sem_w = scr[6 + 2 * n:8 + 2 * n]

        wid = lax.axis_index("s") * NC + lax.axis_index("c")
        base = wid * PER_W

        def gidx(slot, t):
            return idx[slot][1] if use_dst[t] else idx[slot][0]

        def fire(c, slot):
            off = base + c * CB
            pltpu.sync_copy(src_hbm.at[pl.ds(off, CB)], idx[slot][0])
            pltpu.sync_copy(dst_hbm.at[pl.ds(off, CB)], idx[slot][1])
            for t in range(n):
                pltpu.async_copy(tabs[t].at[gidx(slot, t)], bufs[slot][t],
                                 sem_g[slot])

        def complete(c, slot):
            off = base + c * CB
            for t in range(n):
                pltpu.make_async_copy(tabs[t].at[gidx(slot, t)],
                                      bufs[slot][t], sem_g[slot]).wait()
            if combine:
                for t in range(1, n):
                    _vec_binop(bufs[slot][0], bufs[slot][t],
                               lambda a, b: a + b)
            for t in range(n_out):
                pltpu.async_copy(bufs[slot][t], outs[t].at[pl.ds(off, CB)],
                                 sem_w[slot])

        def wait_reuse(c, slot):
            off = base + c * CB
            for t in range(n_out):
                pltpu.make_async_copy(bufs[slot][t],
                                      outs[t].at[pl.ds(off, CB)],
                                      sem_w[slot]).wait()

        _pipeline(NCH, fire, complete, wait_reuse)

    return gather


@functools.cache
def _sc_kernels():
    mesh = plsc.VectorSubcoreMesh(core_axis_name="c", subcore_axis_name="s")

    gather_qk = _make_gather(mesh, (True, False), combine=False)
    gather_ab = _make_gather(mesh, (False, True), combine=True)

    @functools.partial(
        pl.kernel,
        mesh=mesh,
        out_type=[jax.ShapeDtypeStruct((NC, NPAD, D), jnp.float32)] * 2,
        scratch_types=(
            [pltpu.VMEM((CB,), jnp.int32)] * 4
            + [pltpu.VMEM((CB, D), jnp.float32)] * 4
            + [pltpu.VMEM_SHARED((NPAD, D), jnp.float32)]
            + [pltpu.SemaphoreType.DMA] * 4
        ),
    )
    def scatter(src_hbm, dst_hbm, v_hbm, exx_hbm, zero_hbm, on_out, den_out,
                is0, is1, id0, id1, bv0, bv1, be0, be1, sh,
                sem0, sem1, ssc0, ssc1):
        cid = lax.axis_index("c")
        sid = lax.axis_index("s")
        wid = sid * NC + cid
        r0 = sid * ROWS_PER_SUB
        base = wid * PER_W
        idx_s = (is0, is1)
        idx_d = (id0, id1)
        bv = (bv0, bv1)
        be = (be0, be1)
        sems = (sem0, sem1)
        sems_sc = (ssc0, ssc1)

        def zero_init():
            pltpu.sync_copy(zero_hbm.at[pl.ds(r0, ROWS_PER_SUB)],
                            sh.at[pl.ds(r0, ROWS_PER_SUB)])
            plsc.subcore_barrier()

        def drain(out_hbm):
            plsc.subcore_barrier()
            pltpu.sync_copy(sh.at[pl.ds(r0, ROWS_PER_SUB)],
                            out_hbm.at[cid, pl.ds(r0, ROWS_PER_SUB)])
            plsc.subcore_barrier()

        zero_init()

        def fire1(c, slot):
            off = base + c * CB
            pltpu.sync_copy(src_hbm.at[pl.ds(off, CB)], idx_s[slot])
            pltpu.sync_copy(dst_hbm.at[pl.ds(off, CB)], idx_d[slot])
            pltpu.async_copy(v_hbm.at[idx_s[slot]], bv[slot], sems[slot])
            pltpu.async_copy(exx_hbm.at[pl.ds(off, CB)], be[slot],
                             sems[slot])

        def complete1(c, slot):
            off = base + c * CB
            pltpu.make_async_copy(v_hbm.at[idx_s[slot]], bv[slot],
                                  sems[slot]).wait()
            pltpu.make_async_copy(exx_hbm.at[pl.ds(off, CB)], be[slot],
                                  sems[slot]).wait()
            _vec_binop(bv[slot], be[slot], lambda a, b: a * b)
            pltpu.async_copy(bv[slot], sh.at[idx_d[slot]], sems_sc[slot],
                             add=True)

        def reuse1(c, slot):
            pltpu.make_async_copy(bv[slot], sh.at[idx_d[slot]],
                                  sems_sc[slot]).wait()

        _pipeline(NCH, fire1, complete1, reuse1)
        drain(on_out)

        zero_init()

        def fire2(c, slot):
            off = base + c * CB
            pltpu.async_copy(dst_hbm.at[pl.ds(off, CB)], idx_d[slot],
                             sems[slot])
            pltpu.async_copy(exx_hbm.at[pl.ds(off, CB)], be[slot],
                             sems[slot])

        def complete2(c, slot):
            off = base + c * CB
            pltpu.make_async_copy(dst_hbm.at[pl.ds(off, CB)], idx_d[slot],
                                  sems[slot]).wait()
            pltpu.make_async_copy(exx_hbm.at[pl.ds(off, CB)], be[slot],
                                  sems[slot]).wait()
            pltpu.async_copy(be[slot], sh.at[idx_d[slot]], sems_sc[slot],
                             add=True)

        def reuse2(c, slot):
            pltpu.make_async_copy(be[slot], sh.at[idx_d[slot]],
                                  sems_sc[slot]).wait()

        _pipeline(NCH, fire2, complete2, reuse2)
        drain(den_out)

    return gather_qk, gather_ab, scatter


def kernel(h_node, h_edge, edge_index, params):
    p = params
    gqk, gab, sct = _sc_kernels()
    ei = edge_index.astype(jnp.int32)
    e_src, e_dst = ei[0], ei[1]
    expm = jnp.asarray(_EXP_NP)
    hs = jnp.asarray(_EXP_NP.T)
    r2 = lambda t: t.reshape(1, -1)
    wspec = lambda shp: pl.BlockSpec(shp, lambda i: (0, 0))

    q, k, v = pl.pallas_call(
        _tc_qkv_body,
        out_shape=[jax.ShapeDtypeStruct((N, D), jnp.float32)] * 3,
    )(h_node, p["q"]["W"], r2(p["q"]["b"]), p["k"]["W"], r2(p["k"]["b"]),
      p["v"]["W"], r2(p["v"]["b"]))

    eb = pl.pallas_call(
        _tc_eb_body,
        grid=(GE,),
        in_specs=[pl.BlockSpec((EBLK, D), lambda i: (i, 0)),
                  pl.BlockSpec((D, H), lambda i: (0, 0)),
                  pl.BlockSpec((1, H), lambda i: (0, 0))],
        out_specs=pl.BlockSpec((EBLK, H), lambda i: (i, 0)),
        out_shape=jax.ShapeDtypeStruct((E, H), jnp.float32),
    )(h_edge, p["eb"]["W"], r2(p["eb"]["b"]))

    qd, ks = gqk(q, k, e_src, e_dst)

    exx = pl.pallas_call(
        _tc_msg_body,
        grid=(GE,),
        in_specs=[pl.BlockSpec((EBLK, D), lambda i: (i, 0)),
                  pl.BlockSpec((EBLK, D), lambda i: (i, 0)),
                  pl.BlockSpec((EBLK, H), lambda i: (i, 0)),
                  pl.BlockSpec((D, H), lambda i: (0, 0)),
                  pl.BlockSpec((H, D), lambda i: (0, 0))],
        out_specs=pl.BlockSpec((EBLK, D), lambda i: (i, 0)),
        out_shape=jax.ShapeDtypeStruct((E, D), jnp.float32),
    )(qd, ks, eb, hs, expm)

    zero = jnp.zeros((NPAD, D), jnp.float32)
    on_p, den_p = sct(e_src, e_dst, v, exx, zero)

    hn, a_tab, b_tab = pl.pallas_call(
        _tc_node_body,
        grid=(GN,),
        in_specs=[pl.BlockSpec((NB, D), lambda i: (i, 0))] * 5 +
                 [wspec((D, D)), wspec((1, D)),
                  wspec((1, D)), wspec((1, D)),
                  wspec((D, 2 * D)), wspec((1, 2 * D)),
                  wspec((2 * D, D)), wspec((1, D)),
                  wspec((1, D)), wspec((1, D)),
                  wspec((D, D)), wspec((1, D)), wspec((D, D))],
        out_specs=[pl.BlockSpec((NB, D), lambda i: (i, 0))] * 3,
        out_shape=[jax.ShapeDtypeStruct((N, D), jnp.float32)] * 3,
    )(h_node, on_p[0], on_p[1], den_p[0], den_p[1],
      p["o"]["W"], r2(p["o"]["b"]),
      r2(p["ln1"]["g"]), r2(p["ln1"]["b"]),
      p["ffn1"]["W"], r2(p["ffn1"]["b"]),
      p["ffn2"]["W"], r2(p["ffn2"]["b"]),
      r2(p["ln2"]["g"]), r2(p["ln2"]["b"]),
      p["eu1"]["W"][:D], r2(p["eu1"]["b"]), p["eu1"]["W"][D:2 * D])

    res = gab(a_tab, b_tab, e_src, e_dst)
    anbn = res[0] if isinstance(res, (list, tuple)) else res

    he = pl.pallas_call(
        _tc_edge_body,
        grid=(GE,),
        in_specs=[pl.BlockSpec((EBLK, D), lambda i: (i, 0)),
                  pl.BlockSpec((EBLK, D), lambda i: (i, 0)),
                  wspec((D, D)), wspec((D, D)), wspec((1, D)),
                  wspec((1, D)), wspec((1, D))],
        out_specs=pl.BlockSpec((EBLK, D), lambda i: (i, 0)),
        out_shape=jax.ShapeDtypeStruct((E, D), jnp.float32),
    )(anbn, h_edge, p["eu1"]["W"][2 * D:], p["eu2"]["W"], r2(p["eu2"]["b"]),
      r2(p["lne"]["g"]), r2(p["lne"]["b"]))

    return hn, he


# speedup vs baseline: 1.0749x; 1.0481x over previous
import functools

import numpy as np
import jax
import jax.numpy as jnp
from jax import lax
from jax.experimental import pallas as pl
from jax.experimental.pallas import tpu as pltpu
from jax.experimental.pallas import tpu_sc as plsc

N = 10000
E = 320000
D = 128
H = 8
DH = D // H
SCALE = DH ** -0.5

NC = 2
NS = 16
NW = NC * NS
PER_W = E // NW
CB = 80
NCH = PER_W // CB
NPAD = 10240
ROWS_PER_SUB = NPAD // NS

EBLK = 2560
GE = E // EBLK
NB = 400
GN = N // NB

_EXP_NP = np.kron(np.eye(H, dtype=np.float32), np.ones((1, DH), np.float32))


def _ln(x, g, b, eps=1e-5):
    m = jnp.mean(x, axis=-1, keepdims=True)
    v = jnp.mean((x - m) ** 2, axis=-1, keepdims=True)
    return (x - m) / jnp.sqrt(v + eps) * g + b


def _gelu(x):
    return x * 0.5 * (1.0 + lax.erf(x * np.float32(1.0 / np.sqrt(2.0))))


def _dot(a, b):
    return jnp.dot(a, b, preferred_element_type=jnp.float32)


def _tc_qkv_body(x_ref, wq, bq, wk, bk, wv, bv, q_out, k_out, v_out):
    x = x_ref[...]
    q_out[...] = _dot(x, wq[...]) + bq[...]
    k_out[...] = _dot(x, wk[...]) + bk[...]
    v_out[...] = _dot(x, wv[...]) + bv[...]


def _tc_eb_body(he_ref, w, b, eb_out):
    eb_out[...] = _dot(he_ref[...], w[...]) + b[...]


def _tc_msg_body(qd, ks, eb, hs, expm, exx_out):
    s = _dot(qd[...] * ks[...], hs[...]) * SCALE + eb[...]
    exx_out[...] = _dot(jnp.exp(s), expm[...])


def _tc_node_body(hnode, on0, on1, den0, den1,
                  wo, bo, g1, b1, wf1, bf1, wf2, bf2, g2, b2,
                  w1a, b1u, w1b, hn_out, a_out, b_out):
    agg = (on0[...] + on1[...]) / (den0[...] + den1[...] + 1e-16)
    out_node = _dot(agg, wo[...]) + bo[...]
    h1 = _ln(hnode[...] + out_node, g1[...], b1[...])
    ff = _dot(_gelu(_dot(h1, wf1[...]) + bf1[...]), wf2[...]) + bf2[...]
    hn = _ln(h1 + ff, g2[...], b2[...])
    hn_out[...] = hn
    a_out[...] = _dot(hn, w1a[...]) + b1u[...]
    b_out[...] = _dot(hn, w1b[...])


def _tc_edge_body(anbn, he, w1c, w2, b2, ge, be, he_out):
    t = anbn[...] + _dot(he[...], w1c[...])
    hen = _dot(_gelu(t), w2[...]) + b2[...]
    he_out[...] = _ln(he[...] + hen, ge[...], be[...])


def _pipeline(nch, fire, complete, wait_reuse):
    p = nch // 2
    odd = nch % 2 == 1
    fire(0, 0)
    fire(1, 1)

    def body(j, carry):
        c0 = 2 * j
        complete(c0, 0)
        complete(c0 + 1, 1)
        wait_reuse(c0, 0)
        if odd:
            fire(c0 + 2, 0)
        else:
            @pl.when(j < p - 1)
            def _pf0():
                fire(c0 + 2, 0)
        wait_reuse(c0 + 1, 1)

        @pl.when(j < p - 1)
        def _pf1():
            fire(c0 + 3, 1)

        return carry

    lax.fori_loop(0, p, body, 0)
    if odd:
        complete(nch - 1, 0)
        wait_reuse(nch - 1, 0)


def _vec_binop(dst, src, op):
    def row(i, carry):
        for r in range(D // 16):
            sl = pl.ds(r * 16, 16)
            dst[i, sl] = op(dst[i, sl], src[i, sl])
        return carry

    lax.fori_loop(0, CB, row, 0)


def _make_gather(mesh, use_dst, combine):
    n = len(use_dst)
    n_out = 1 if combine else n

    @functools.partial(
        pl.kernel,
        mesh=mesh,
        out_type=[jax.ShapeDtypeStruct((E, D), jnp.float32)] * n_out,
        scratch_types=(
            [pltpu.VMEM((PER_W,), jnp.int32)] * 2
            + [pltpu.VMEM((CB, D), jnp.float32)] * (2 * n)
            + [pltpu.SemaphoreType.DMA] * 4
        ),
    )
    def gather(*refs):
        tabs = refs[:n]
        src_hbm, dst_hbm = refs[n], refs[n + 1]
        outs = refs[n + 2:n + 2 + n_out]
        scr = refs[n + 2 + n_out:]
        idx_all = scr[0:2]
        bufs = (scr[2:2 + n], scr[2 + n:2 + 2 * n])
        sem_g = scr[2 + 2 * n:4 + 2 * n]
        sem_w = scr[4 + 2 * n:6 + 2 * n]

        wid = lax.axis_index("s") * NC + lax.axis_index("c")
        base = wid * PER_W

        pltpu.sync_copy(src_hbm.at[pl.ds(base, PER_W)], idx_all[0])
        pltpu.sync_copy(dst_hbm.at[pl.ds(base, PER_W)], idx_all[1])

        def gidx(c, t):
            a = idx_all[1] if use_dst[t] else idx_all[0]
            return a.at[pl.ds(c * CB, CB)]

        def fire(c, slot):
            for t in range(n):
                pltpu.async_copy(tabs[t].at[gidx(c, t)], bufs[slot][t],
                                 sem_g[slot])

        def complete(c, slot):
            off = base + c * CB
            for t in range(n):
                pltpu.make_async_copy(tabs[t].at[gidx(c, t)],
                                      bufs[slot][t], sem_g[slot]).wait()
            if combine:
                for t in range(1, n):
                    _vec_binop(bufs[slot][0], bufs[slot][t],
                               lambda a, b: a + b)
            for t in range(n_out):
                pltpu.async_copy(bufs[slot][t], outs[t].at[pl.ds(off, CB)],
                                 sem_w[slot])

        def wait_reuse(c, slot):
            off = base + c * CB
            for t in range(n_out):
                pltpu.make_async_copy(bufs[slot][t],
                                      outs[t].at[pl.ds(off, CB)],
                                      sem_w[slot]).wait()

        _pipeline(NCH, fire, complete, wait_reuse)

    return gather


@functools.cache
def _sc_kernels():
    mesh = plsc.VectorSubcoreMesh(core_axis_name="c", subcore_axis_name="s")

    gather_qk = _make_gather(mesh, (True, False), combine=False)
    gather_ab = _make_gather(mesh, (False, True), combine=True)

    @functools.partial(
        pl.kernel,
        mesh=mesh,
        out_type=[jax.ShapeDtypeStruct((NC, NPAD, D), jnp.float32)] * 2,
        scratch_types=(
            [pltpu.VMEM((CB,), jnp.int32)] * 4
            + [pltpu.VMEM((CB, D), jnp.float32)] * 4
            + [pltpu.VMEM_SHARED((NPAD, D), jnp.float32)]
            + [pltpu.SemaphoreType.DMA] * 4
        ),
    )
    def scatter(src_hbm, dst_hbm, v_hbm, exx_hbm, zero_hbm, on_out, den_out,
                is0, is1, id0, id1, bv0, bv1, be0, be1, sh,
                sem0, sem1, ssc0, ssc1):
        cid = lax.axis_index("c")
        sid = lax.axis_index("s")
        wid = sid * NC + cid
        r0 = sid * ROWS_PER_SUB
        base = wid * PER_W
        idx_s = (is0, is1)
        idx_d = (id0, id1)
        bv = (bv0, bv1)
        be = (be0, be1)
        sems = (sem0, sem1)
        sems_sc = (ssc0, ssc1)

        def zero_init():
            pltpu.sync_copy(zero_hbm.at[pl.ds(r0, ROWS_PER_SUB)],
                            sh.at[pl.ds(r0, ROWS_PER_SUB)])
            plsc.subcore_barrier()

        def drain(out_hbm):
            plsc.subcore_barrier()
            pltpu.sync_copy(sh.at[pl.ds(r0, ROWS_PER_SUB)],
                            out_hbm.at[cid, pl.ds(r0, ROWS_PER_SUB)])
            plsc.subcore_barrier()

        zero_init()

        def fire1(c, slot):
            off = base + c * CB
            pltpu.sync_copy(src_hbm.at[pl.ds(off, CB)], idx_s[slot])
            pltpu.sync_copy(dst_hbm.at[pl.ds(off, CB)], idx_d[slot])
            pltpu.async_copy(v_hbm.at[idx_s[slot]], bv[slot], sems[slot])
            pltpu.async_copy(exx_hbm.at[pl.ds(off, CB)], be[slot],
                             sems[slot])

        def complete1(c, slot):
            off = base + c * CB
            pltpu.make_async_copy(v_hbm.at[idx_s[slot]], bv[slot],
                                  sems[slot]).wait()
            pltpu.make_async_copy(exx_hbm.at[pl.ds(off, CB)], be[slot],
                                  sems[slot]).wait()
            _vec_binop(bv[slot], be[slot], lambda a, b: a * b)
            pltpu.async_copy(bv[slot], sh.at[idx_d[slot]], sems_sc[slot],
                             add=True)

        def reuse1(c, slot):
            pltpu.make_async_copy(bv[slot], sh.at[idx_d[slot]],
                                  sems_sc[slot]).wait()

        _pipeline(NCH, fire1, complete1, reuse1)
        drain(on_out)

        zero_init()

        def fire2(c, slot):
            off = base + c * CB
            pltpu.async_copy(dst_hbm.at[pl.ds(off, CB)], idx_d[slot],
                             sems[slot])
            pltpu.async_copy(exx_hbm.at[pl.ds(off, CB)], be[slot],
                             sems[slot])

        def complete2(c, slot):
            off = base + c * CB
            pltpu.make_async_copy(dst_hbm.at[pl.ds(off, CB)], idx_d[slot],
                                  sems[slot]).wait()
            pltpu.make_async_copy(exx_hbm.at[pl.ds(off, CB)], be[slot],
                                  sems[slot]).wait()
            pltpu.async_copy(be[slot], sh.at[idx_d[slot]], sems_sc[slot],
                             add=True)

        def reuse2(c, slot):
            pltpu.make_async_copy(be[slot], sh.at[idx_d[slot]],
                                  sems_sc[slot]).wait()

        _pipeline(NCH, fire2, complete2, reuse2)
        drain(den_out)

    return gather_qk, gather_ab, scatter


def kernel(h_node, h_edge, edge_index, params):
    p = params
    gqk, gab, sct = _sc_kernels()
    ei = edge_index.astype(jnp.int32)
    e_src, e_dst = ei[0], ei[1]
    expm = jnp.asarray(_EXP_NP)
    hs = jnp.asarray(_EXP_NP.T)
    r2 = lambda t: t.reshape(1, -1)
    wspec = lambda shp: pl.BlockSpec(shp, lambda i: (0, 0))

    q, k, v = pl.pallas_call(
        _tc_qkv_body,
        out_shape=[jax.ShapeDtypeStruct((N, D), jnp.float32)] * 3,
    )(h_node, p["q"]["W"], r2(p["q"]["b"]), p["k"]["W"], r2(p["k"]["b"]),
      p["v"]["W"], r2(p["v"]["b"]))

    eb = pl.pallas_call(
        _tc_eb_body,
        grid=(GE,),
        in_specs=[pl.BlockSpec((EBLK, D), lambda i: (i, 0)),
                  pl.BlockSpec((D, H), lambda i: (0, 0)),
                  pl.BlockSpec((1, H), lambda i: (0, 0))],
        out_specs=pl.BlockSpec((EBLK, H), lambda i: (i, 0)),
        out_shape=jax.ShapeDtypeStruct((E, H), jnp.float32),
    )(h_edge, p["eb"]["W"], r2(p["eb"]["b"]))

    qd, ks = gqk(q, k, e_src, e_dst)

    exx = pl.pallas_call(
        _tc_msg_body,
        grid=(GE,),
        in_specs=[pl.BlockSpec((EBLK, D), lambda i: (i, 0)),
                  pl.BlockSpec((EBLK, D), lambda i: (i, 0)),
                  pl.BlockSpec((EBLK, H), lambda i: (i, 0)),
                  pl.BlockSpec((D, H), lambda i: (0, 0)),
                  pl.BlockSpec((H, D), lambda i: (0, 0))],
        out_specs=pl.BlockSpec((EBLK, D), lambda i: (i, 0)),
        out_shape=jax.ShapeDtypeStruct((E, D), jnp.float32),
    )(qd, ks, eb, hs, expm)

    zero = jnp.zeros((NPAD, D), jnp.float32)
    on_p, den_p = sct(e_src, e_dst, v, exx, zero)

    hn, a_tab, b_tab = pl.pallas_call(
        _tc_node_body,
        grid=(GN,),
        in_specs=[pl.BlockSpec((NB, D), lambda i: (i, 0))] * 5 +
                 [wspec((D, D)), wspec((1, D)),
                  wspec((1, D)), wspec((1, D)),
                  wspec((D, 2 * D)), wspec((1, 2 * D)),
                  wspec((2 * D, D)), wspec((1, D)),
                  wspec((1, D)), wspec((1, D)),
                  wspec((D, D)), wspec((1, D)), wspec((D, D))],
        out_specs=[pl.BlockSpec((NB, D), lambda i: (i, 0))] * 3,
        out_shape=[jax.ShapeDtypeStruct((N, D), jnp.float32)] * 3,
    )(h_node, on_p[0], on_p[1], den_p[0], den_p[1],
      p["o"]["W"], r2(p["o"]["b"]),
      r2(p["ln1"]["g"]), r2(p["ln1"]["b"]),
      p["ffn1"]["W"], r2(p["ffn1"]["b"]),
      p["ffn2"]["W"], r2(p["ffn2"]["b"]),
      r2(p["ln2"]["g"]), r2(p["ln2"]["b"]),
      p["eu1"]["W"][:D], r2(p["eu1"]["b"]), p["eu1"]["W"][D:2 * D])

    res = gab(a_tab, b_tab, e_src, e_dst)
    anbn = res[0] if isinstance(res, (list, tuple)) else res

    he = pl.pallas_call(
        _tc_edge_body,
        grid=(GE,),
        in_specs=[pl.BlockSpec((EBLK, D), lambda i: (i, 0)),
                  pl.BlockSpec((EBLK, D), lambda i: (i, 0)),
                  wspec((D, D)), wspec((D, D)), wspec((1, D)),
                  wspec((1, D)), wspec((1, D))],
        out_specs=pl.BlockSpec((EBLK, D), lambda i: (i, 0)),
        out_shape=jax.ShapeDtypeStruct((E, D), jnp.float32),
    )(anbn, h_edge, p["eu1"]["W"][2 * D:], p["eu2"]["W"], r2(p["eu2"]["b"]),
      r2(p["lne"]["g"]), r2(p["lne"]["b"]))

    return hn, he
